# baseline (speedup 1.0000x reference)
import functools

import jax
import jax.numpy as jnp
from jax import lax
from jax.experimental import pallas as pl
from jax.experimental.pallas import tpu as pltpu
from jax.experimental.pallas import tpu_sc as plsc

B = 16
H = 16
C = 128
W = 512
OV = 4
NVQ = 8
CD = 32
K = 1024
FIX = H * C
INVQ = FIX * OV // NVQ
T = W // OV
GD = NVQ * CD
NTOK = B * NVQ * T
DEF = lax.Precision.DEFAULT
HIGH = lax.Precision.HIGHEST


def _stage1_body(x_ref, wdbig_ref, cb_ref, psel_ref,
                 zn_ref, code_ref, gidx_ref, en_ref):
    x = x_ref[0]
    psel = psel_ref[...]

    s = lax.dot_general(x[0], wdbig_ref[0], (((1,), (0,)), ((), ())),
                        precision=DEF)
    for h in range(1, H):
        s = s + lax.dot_general(x[h], wdbig_ref[h], (((1,), (0,)), ((), ())),
                                precision=DEF)

    zsel = lax.dot_general(psel, s, (((1,), (0,)), ((), ())),
                           precision=HIGH)

    zn_list, code_list, gidx_list = [], [], []
    iota = lax.broadcasted_iota(jnp.int32, (T, K), 1)
    first = pl.program_id(0) == 0
    for g in range(NVQ):
        ov = g // 2
        zd = zsel[ov * T:(ov + 1) * T, g * CD:(g + 1) * CD]
        nrm = jnp.sqrt(jnp.sum(zd * zd, axis=-1, keepdims=True))
        zn = zd / (nrm + 1e-8)

        emb = cb_ref[g]
        enrm = jnp.sqrt(jnp.sum(emb * emb, axis=-1, keepdims=True))
        en = emb / (enrm + 1e-8)
        ensq = jnp.sum(en * en, axis=-1)
        znsq = jnp.sum(zn * zn, axis=-1, keepdims=True)

        @pl.when(first)
        def _():
            en_ref[g * K:(g + 1) * K, :] = jnp.concatenate(
                [en, jnp.zeros((K, C - CD), jnp.float32)], axis=1)

        dots = lax.dot_general(zn, en, (((1,), (1,)), ((), ())), precision=DEF)
        d = znsq - 2.0 * dots + ensq[None, :]
        dmin = jnp.min(d, axis=-1, keepdims=True)
        code = jnp.min(jnp.where(d == dmin, iota, K), axis=-1)
        zn_list.append(zn)
        code_list.append(code)
        gidx_list.append(code + g * K)

    zn_ref[0] = jnp.stack(zn_list, axis=0)
    code_ref[0] = jnp.stack(code_list, axis=0)
    gidx_ref[0] = jnp.stack(gidx_list, axis=0)


def _stage2_body(q_ref, zn_ref, wubig_ref, psel_ref, zq_ref, cm_ref):
    psel = psel_ref[...]
    zero = jnp.zeros((T, CD), jnp.float32)
    rows = []
    cm_part = jnp.zeros((), jnp.float32)
    for ov in range(OV):
        pieces = []
        for g in range(NVQ):
            if g // 2 == ov:
                qd = q_ref[0, g][:, :CD]
                diff = zn_ref[0, g] - qd
                cm_part = cm_part + jnp.sum(diff * diff)
                pieces.append(qd)
            else:
                pieces.append(zero)
        rows.append(jnp.concatenate(pieces, axis=1))
    qstack = jnp.concatenate(rows, axis=0)

    q = lax.dot_general(psel, qstack, (((0,), (0,)), ((), ())),
                        precision=HIGH)

    for h in range(H):
        zq_ref[0, h] = lax.dot_general(q, wubig_ref[h], (((1,), (0,)), ((), ())),
                                       precision=DEF)

    @pl.when(pl.program_id(0) == 0)
    def _():
        cm_ref[...] = jnp.zeros((1, 1), jnp.float32)

    cm_ref[...] += jnp.reshape(cm_part, (1, 1))


@functools.partial(jax.jit)
def _stage1(z4, wdbig, codebooks, psel):
    out_shapes = (
        jax.ShapeDtypeStruct((B, NVQ, T, CD), jnp.float32),
        jax.ShapeDtypeStruct((B, NVQ, T), jnp.int32),
        jax.ShapeDtypeStruct((B, NVQ, T), jnp.int32),
        jax.ShapeDtypeStruct((NVQ * K, C), jnp.float32),
    )
    return pl.pallas_call(
        _stage1_body,
        grid=(B,),
        in_specs=[
            pl.BlockSpec((1, H, W, C), lambda b: (b, 0, 0, 0)),
            pl.BlockSpec((H, C, GD), lambda b: (0, 0, 0)),
            pl.BlockSpec((NVQ, K, CD), lambda b: (0, 0, 0)),
            pl.BlockSpec((W, W), lambda b: (0, 0)),
        ],
        out_specs=(
            pl.BlockSpec((1, NVQ, T, CD), lambda b: (b, 0, 0, 0)),
            pl.BlockSpec((1, NVQ, T), lambda b: (b, 0, 0)),
            pl.BlockSpec((1, NVQ, T), lambda b: (b, 0, 0)),
            pl.BlockSpec((NVQ * K, C), lambda b: (0, 0)),
        ),
        out_shape=out_shapes,
    )(z4, wdbig, codebooks, psel)


def _make_sc_gather():
    info = plsc.get_sparse_core_info()
    nc, ns = info.num_cores, info.num_subcores
    nw = nc * ns
    assert NTOK % nw == 0
    b_per_w = NTOK // nw
    mesh = plsc.VectorSubcoreMesh(core_axis_name="c", subcore_axis_name="s")

    @functools.partial(
        pl.kernel, mesh=mesh,
        out_type=jax.ShapeDtypeStruct((NTOK, C), jnp.float32),
        scratch_types=[
            pltpu.VMEM((b_per_w,), jnp.int32),
            pltpu.VMEM((b_per_w, C), jnp.float32),
            pltpu.SemaphoreType.DMA,
        ],
    )
    def sc_gather(table_hbm, idx_hbm, out_hbm, idx_v, rows_v, sem):
        wid = lax.axis_index("s") * nc + lax.axis_index("c")
        base = wid * b_per_w
        pltpu.sync_copy(idx_hbm.at[pl.ds(base, b_per_w)], idx_v)
        pltpu.async_copy(table_hbm.at[idx_v], rows_v, sem).wait()
        pltpu.sync_copy(rows_v, out_hbm.at[pl.ds(base, b_per_w)])

    return sc_gather


_sc_gather = _make_sc_gather()


@functools.partial(jax.jit)
def _stage2(q_all, zn_out, wubig, psel):
    out_shapes = (
        jax.ShapeDtypeStruct((B, H, W, C), jnp.float32),
        jax.ShapeDtypeStruct((1, 1), jnp.float32),
    )
    return pl.pallas_call(
        _stage2_body,
        grid=(B,),
        in_specs=[
            pl.BlockSpec((1, NVQ, T, C), lambda b: (b, 0, 0, 0)),
            pl.BlockSpec((1, NVQ, T, CD), lambda b: (b, 0, 0, 0)),
            pl.BlockSpec((H, GD, C), lambda b: (0, 0, 0)),
            pl.BlockSpec((W, W), lambda b: (0, 0)),
        ],
        out_specs=(
            pl.BlockSpec((1, H, W, C), lambda b: (b, 0, 0, 0)),
            pl.BlockSpec((1, 1), lambda b: (0, 0)),
        ),
        out_shape=out_shapes,
    )(q_all, zn_out, wubig, psel)


def _prep_weights(W_down, W_up):
    wd4 = W_down.reshape(NVQ, CD, 64, H).transpose(3, 0, 2, 1)
    p = wd4.transpose(0, 2, 1, 3).reshape(H, 64, GD)
    m = jnp.repeat((jnp.arange(NVQ) % 2 == 0), CD).astype(jnp.float32)
    wdbig = jnp.concatenate([p * m[None, None, :],
                             p * (1.0 - m)[None, None, :]], axis=1)

    wu4 = W_up.reshape(NVQ, 64, H, CD).transpose(2, 0, 3, 1)
    quu = wu4.reshape(H, GD, 64)
    wubig = jnp.concatenate([quu * m[None, :, None],
                             quu * (1.0 - m)[None, :, None]], axis=2)
    return wdbig, wubig


def kernel(z_e, W_down, W_up, codebooks):
    z4 = z_e.reshape(B, H, W, C)
    wdbig, wubig = _prep_weights(W_down, W_up)
    tt = jnp.arange(W) // OV
    ovv = jnp.arange(W) % OV
    psel = jnp.zeros((W, W), jnp.float32).at[ovv * T + tt, jnp.arange(W)].set(1.0)

    zn_out, codes, gidx, en_table = _stage1(z4, wdbig, codebooks, psel)
    q_flat = _sc_gather(en_table, gidx.reshape(NTOK))
    q_all = q_flat.reshape(B, NVQ, T, C)
    zq4, cmsum = _stage2(q_all, zn_out, wubig, psel)

    zq = zq4.reshape(B, H * W, C)
    cm = cmsum[0, 0] / (NVQ * B * T * CD)
    return (zq, zn_out, codes, cm, cm)


# speedup vs baseline: 1.1491x; 1.1491x over previous
import functools

import jax
import jax.numpy as jnp
from jax import lax
from jax.experimental import pallas as pl
from jax.experimental.pallas import tpu as pltpu
from jax.experimental.pallas import tpu_sc as plsc

B = 16
H = 16
C = 128
W = 512
OV = 4
NVQ = 8
CD = 32
K = 1024
FIX = H * C
INVQ = FIX * OV // NVQ
T = W // OV
GD = NVQ * CD
NTOK = B * NVQ * T
DEF = lax.Precision.DEFAULT
HIGH = lax.Precision.HIGHEST


def _prologue_body(cb_ref, en_ref, ensq_ref):
    for g in range(NVQ):
        emb = cb_ref[g]
        enrm = jnp.sqrt(jnp.sum(emb * emb, axis=-1, keepdims=True))
        en = emb / (enrm + 1e-8)
        en_ref[g * K:(g + 1) * K, :] = jnp.concatenate(
            [en, jnp.zeros((K, C - CD), jnp.float32)], axis=1)
        ensq_ref[g] = jnp.sum(en * en, axis=-1)


def _stage1_body(x_ref, wdbig_ref, en_ref, ensq_ref, psel_ref,
                 zn_ref, code_ref, gidx_ref):
    x = x_ref[0]
    psel = psel_ref[...]

    s = lax.dot_general(x[0], wdbig_ref[0], (((1,), (0,)), ((), ())),
                        precision=DEF)
    for h in range(1, H):
        s = s + lax.dot_general(x[h], wdbig_ref[h], (((1,), (0,)), ((), ())),
                                precision=DEF)

    zsel = lax.dot_general(psel, s, (((1,), (0,)), ((), ())),
                           precision=HIGH)

    zn_list, code_list, gidx_list = [], [], []
    iota = lax.broadcasted_iota(jnp.int32, (T, K), 1)
    for g in range(NVQ):
        ov = g // 2
        zd = zsel[ov * T:(ov + 1) * T, g * CD:(g + 1) * CD]
        nrm = jnp.sqrt(jnp.sum(zd * zd, axis=-1, keepdims=True))
        zn = zd / (nrm + 1e-8)

        en = en_ref[g * K:(g + 1) * K, :CD]
        ensq = ensq_ref[g]
        znsq = jnp.sum(zn * zn, axis=-1, keepdims=True)

        dots = lax.dot_general(zn, en, (((1,), (1,)), ((), ())), precision=DEF)
        d = znsq - 2.0 * dots + ensq[None, :]
        dmin = jnp.min(d, axis=-1, keepdims=True)
        code = jnp.min(jnp.where(d == dmin, iota, K), axis=-1)
        zn_list.append(zn)
        code_list.append(code)
        gidx_list.append(code + g * K)

    zn_ref[0] = jnp.stack(zn_list, axis=0)
    code_ref[0] = jnp.stack(code_list, axis=0)
    gidx_ref[0] = jnp.stack(gidx_list, axis=0)


def _stage2_body(q_ref, zn_ref, wubig_ref, psel_ref, zq_ref, cm_ref):
    psel = psel_ref[...]
    zero = jnp.zeros((T, CD), jnp.float32)
    rows = []
    cm_part = jnp.zeros((), jnp.float32)
    for ov in range(OV):
        pieces = []
        for g in range(NVQ):
            if g // 2 == ov:
                qd = q_ref[0, g][:, :CD]
                diff = zn_ref[0, g] - qd
                cm_part = cm_part + jnp.sum(diff * diff)
                pieces.append(qd)
            else:
                pieces.append(zero)
        rows.append(jnp.concatenate(pieces, axis=1))
    qstack = jnp.concatenate(rows, axis=0)

    q = lax.dot_general(psel, qstack, (((0,), (0,)), ((), ())),
                        precision=HIGH)

    for h in range(H):
        zq_ref[0, h] = lax.dot_general(q, wubig_ref[h], (((1,), (0,)), ((), ())),
                                       precision=DEF)

    @pl.when(pl.program_id(0) == 0)
    def _():
        cm_ref[...] = jnp.zeros((1, 1), jnp.float32)

    cm_ref[...] += jnp.reshape(cm_part, (1, 1))


@functools.partial(jax.jit)
def _prologue(codebooks):
    out_shapes = (
        jax.ShapeDtypeStruct((NVQ * K, C), jnp.float32),
        jax.ShapeDtypeStruct((NVQ, K), jnp.float32),
    )
    return pl.pallas_call(
        _prologue_body,
        grid=(1,),
        in_specs=[pl.BlockSpec((NVQ, K, CD), lambda i: (0, 0, 0))],
        out_specs=(
            pl.BlockSpec((NVQ * K, C), lambda i: (0, 0)),
            pl.BlockSpec((NVQ, K), lambda i: (0, 0)),
        ),
        out_shape=out_shapes,
    )(codebooks)


@functools.partial(jax.jit)
def _stage1(z4, wdbig, en_table, ensq, psel):
    out_shapes = (
        jax.ShapeDtypeStruct((B, NVQ, T, CD), jnp.float32),
        jax.ShapeDtypeStruct((B, NVQ, T), jnp.int32),
        jax.ShapeDtypeStruct((B, NVQ, T), jnp.int32),
    )
    return pl.pallas_call(
        _stage1_body,
        grid=(B,),
        in_specs=[
            pl.BlockSpec((1, H, W, C), lambda b: (b, 0, 0, 0)),
            pl.BlockSpec((H, C, GD), lambda b: (0, 0, 0)),
            pl.BlockSpec((NVQ * K, C), lambda b: (0, 0)),
            pl.BlockSpec((NVQ, K), lambda b: (0, 0)),
            pl.BlockSpec((W, W), lambda b: (0, 0)),
        ],
        out_specs=(
            pl.BlockSpec((1, NVQ, T, CD), lambda b: (b, 0, 0, 0)),
            pl.BlockSpec((1, NVQ, T), lambda b: (b, 0, 0)),
            pl.BlockSpec((1, NVQ, T), lambda b: (b, 0, 0)),
        ),
        out_shape=out_shapes,
    )(z4, wdbig, en_table, ensq, psel)


def _make_sc_gather():
    info = plsc.get_sparse_core_info()
    nc, ns = info.num_cores, info.num_subcores
    nw = nc * ns
    assert NTOK % nw == 0
    b_per_w = NTOK // nw
    mesh = plsc.VectorSubcoreMesh(core_axis_name="c", subcore_axis_name="s")

    @functools.partial(
        pl.kernel, mesh=mesh,
        out_type=jax.ShapeDtypeStruct((NTOK, C), jnp.float32),
        scratch_types=[
            pltpu.VMEM((b_per_w,), jnp.int32),
            pltpu.VMEM((b_per_w, C), jnp.float32),
            pltpu.SemaphoreType.DMA,
        ],
    )
    def sc_gather(table_hbm, idx_hbm, out_hbm, idx_v, rows_v, sem):
        wid = lax.axis_index("s") * nc + lax.axis_index("c")
        base = wid * b_per_w
        pltpu.sync_copy(idx_hbm.at[pl.ds(base, b_per_w)], idx_v)
        pltpu.async_copy(table_hbm.at[idx_v], rows_v, sem).wait()
        pltpu.sync_copy(rows_v, out_hbm.at[pl.ds(base, b_per_w)])

    return sc_gather


_sc_gather = _make_sc_gather()


@functools.partial(jax.jit)
def _stage2(q_all, zn_out, wubig, psel):
    out_shapes = (
        jax.ShapeDtypeStruct((B, H, W, C), jnp.float32),
        jax.ShapeDtypeStruct((1, 1), jnp.float32),
    )
    return pl.pallas_call(
        _stage2_body,
        grid=(B,),
        in_specs=[
            pl.BlockSpec((1, NVQ, T, C), lambda b: (b, 0, 0, 0)),
            pl.BlockSpec((1, NVQ, T, CD), lambda b: (b, 0, 0, 0)),
            pl.BlockSpec((H, GD, C), lambda b: (0, 0, 0)),
            pl.BlockSpec((W, W), lambda b: (0, 0)),
        ],
        out_specs=(
            pl.BlockSpec((1, H, W, C), lambda b: (b, 0, 0, 0)),
            pl.BlockSpec((1, 1), lambda b: (0, 0)),
        ),
        out_shape=out_shapes,
    )(q_all, zn_out, wubig, psel)


def _prep_weights(W_down, W_up):
    wd4 = W_down.reshape(NVQ, CD, 64, H).transpose(3, 0, 2, 1)
    p = wd4.transpose(0, 2, 1, 3).reshape(H, 64, GD)
    m = jnp.repeat((jnp.arange(NVQ) % 2 == 0), CD).astype(jnp.float32)
    wdbig = jnp.concatenate([p * m[None, None, :],
                             p * (1.0 - m)[None, None, :]], axis=1)

    wu4 = W_up.reshape(NVQ, 64, H, CD).transpose(2, 0, 3, 1)
    quu = wu4.reshape(H, GD, 64)
    wubig = jnp.concatenate([quu * m[None, :, None],
                             quu * (1.0 - m)[None, :, None]], axis=2)
    return wdbig, wubig


def kernel(z_e, W_down, W_up, codebooks):
    z4 = z_e.reshape(B, H, W, C)
    wdbig, wubig = _prep_weights(W_down, W_up)
    tt = jnp.arange(W) // OV
    ovv = jnp.arange(W) % OV
    psel = jnp.zeros((W, W), jnp.float32).at[ovv * T + tt, jnp.arange(W)].set(1.0)

    en_table, ensq = _prologue(codebooks)
    zn_out, codes, gidx = _stage1(z4, wdbig, en_table, ensq, psel)
    q_flat = _sc_gather(en_table, gidx.reshape(NTOK))
    q_all = q_flat.reshape(B, NVQ, T, C)
    zq4, cmsum = _stage2(q_all, zn_out, wubig, psel)

    zq = zq4.reshape(B, H * W, C)
    cm = cmsum[0, 0] / (NVQ * B * T * CD)
    return (zq, zn_out, codes, cm, cm)
